# baseline (speedup 1.0000x reference)
import functools

import jax
import jax.numpy as jnp
from jax.experimental import pallas as pl

_LANE = 128


def _codec_body(groups, sp_ref, e_ref, d_ref, eb_ref, db_ref, out_ref):
    out_ref[...] = jnp.broadcast_to(db_ref[...], out_ref.shape)
    for i, (a0, w, ho) in enumerate(groups):
        sp_r = sp_ref[:, a0:a0 + w]
        e_r = e_ref[:, ho:ho + w]
        tok = jax.lax.dot_general(
            sp_r, e_r, (((1,), (1,)), ((), ())),
            preferred_element_type=jnp.float32)
        tok = tok + eb_ref[i:i + 1, :]
        d_r = d_ref[ho:ho + w, :]
        rec = jax.lax.dot_general(
            tok, d_r, (((1,), (1,)), ((), ())),
            preferred_element_type=jnp.float32)
        out_ref[:, a0:a0 + w] += rec


def kernel(spikes, neuron_regions, eids, enc_w, enc_b, dec_w, dec_b):
    B, T, N = spikes.shape
    M = B * T
    D = enc_w[0].shape[0]
    R = len(enc_w)
    sizes = [wt.shape[1] for wt in enc_w]
    offs = [0]
    for n in sizes:
        offs.append(offs[-1] + n)

    eh, dh, groups = [], [], []
    ho = 0
    for i in range(R):
        off, n = offs[i], sizes[i]
        a0 = (off // _LANE) * _LANE
        a1 = -(-(off + n) // _LANE) * _LANE
        w = a1 - a0
        lo, hi = off - a0, a1 - (off + n)
        eh.append(jnp.pad(enc_w[i], ((0, 0), (lo, hi))))
        dh.append(jnp.pad(dec_w[i], ((lo, hi), (0, 0))))
        groups.append((a0, w, ho))
        ho += w

    sp2 = spikes.reshape(M, N)
    E = jnp.concatenate(eh, axis=1)
    Dc = jnp.concatenate(dh, axis=0)
    EB = jnp.stack(enc_b, axis=0)
    DB = jnp.concatenate(dec_b)[None, :]
    W = E.shape[1]

    TM = 80
    grid = (M // TM,)
    out = pl.pallas_call(
        functools.partial(_codec_body, tuple(groups)),
        grid=grid,
        in_specs=[
            pl.BlockSpec((TM, N), lambda i: (i, 0)),
            pl.BlockSpec((D, W), lambda i: (0, 0)),
            pl.BlockSpec((W, D), lambda i: (0, 0)),
            pl.BlockSpec(EB.shape, lambda i: (0, 0)),
            pl.BlockSpec((1, N), lambda i: (0, 0)),
        ],
        out_specs=pl.BlockSpec((TM, N), lambda i: (i, 0)),
        out_shape=jax.ShapeDtypeStruct((M, N), spikes.dtype),
    )(sp2, E, Dc, EB, DB)
    return out.reshape(B, T, N)


# speedup vs baseline: 1.2031x; 1.2031x over previous
import functools

import jax
import jax.numpy as jnp
from jax.experimental import pallas as pl

_LANE = 128


def _codec_body(groups, sp_ref, e_ref, d_ref, eb_ref, db_ref, out_ref):
    out_ref[...] = jnp.broadcast_to(db_ref[...], out_ref.shape)
    for i, (a0, w, ho) in enumerate(groups):
        sp_r = sp_ref[:, a0:a0 + w]
        e_r = e_ref[:, ho:ho + w]
        tok = jax.lax.dot_general(
            sp_r, e_r, (((1,), (1,)), ((), ())),
            preferred_element_type=jnp.float32)
        tok = tok + eb_ref[i:i + 1, :]
        d_r = d_ref[ho:ho + w, :]
        rec = jax.lax.dot_general(
            tok, d_r, (((1,), (1,)), ((), ())),
            preferred_element_type=jnp.float32)
        out_ref[:, a0:a0 + w] += rec


def kernel(spikes, neuron_regions, eids, enc_w, enc_b, dec_w, dec_b):
    B, T, N = spikes.shape
    M = B * T
    D = enc_w[0].shape[0]
    R = len(enc_w)
    sizes = [wt.shape[1] for wt in enc_w]
    offs = [0]
    for n in sizes:
        offs.append(offs[-1] + n)

    eh, dh, groups = [], [], []
    ho = 0
    for i in range(R):
        off, n = offs[i], sizes[i]
        a0 = (off // _LANE) * _LANE
        a1 = -(-(off + n) // _LANE) * _LANE
        w = a1 - a0
        lo, hi = off - a0, a1 - (off + n)
        eh.append(jnp.pad(enc_w[i], ((0, 0), (lo, hi))))
        dh.append(jnp.pad(dec_w[i], ((lo, hi), (0, 0))))
        groups.append((a0, w, ho))
        ho += w

    sp2 = spikes.reshape(M, N)
    E = jnp.concatenate(eh, axis=1)
    Dc = jnp.concatenate(dh, axis=0)
    EB = jnp.stack(enc_b, axis=0)
    DB = jnp.concatenate(dec_b)[None, :]
    W = E.shape[1]

    TM = 160
    grid = (M // TM,)
    out = pl.pallas_call(
        functools.partial(_codec_body, tuple(groups)),
        grid=grid,
        in_specs=[
            pl.BlockSpec((TM, N), lambda i: (i, 0)),
            pl.BlockSpec((D, W), lambda i: (0, 0)),
            pl.BlockSpec((W, D), lambda i: (0, 0)),
            pl.BlockSpec(EB.shape, lambda i: (0, 0)),
            pl.BlockSpec((1, N), lambda i: (0, 0)),
        ],
        out_specs=pl.BlockSpec((TM, N), lambda i: (i, 0)),
        out_shape=jax.ShapeDtypeStruct((M, N), spikes.dtype),
    )(sp2, E, Dc, EB, DB)
    return out.reshape(B, T, N)


# speedup vs baseline: 1.4721x; 1.2237x over previous
import functools

import jax
import jax.numpy as jnp
from jax.experimental import pallas as pl

_LANE = 128


def _codec_body(groups, sp_ref, e_ref, d_ref, eb_ref, db_ref, out_ref):
    out_ref[...] = jnp.broadcast_to(db_ref[...], out_ref.shape)
    spb = sp_ref[...].astype(jnp.bfloat16)
    for i, (a0, w, ho) in enumerate(groups):
        sp_r = spb[:, a0:a0 + w]
        e_r = e_ref[:, ho:ho + w]
        tok = jax.lax.dot_general(
            sp_r, e_r, (((1,), (1,)), ((), ())),
            preferred_element_type=jnp.float32)
        tok = (tok + eb_ref[i:i + 1, :]).astype(jnp.bfloat16)
        d_r = d_ref[ho:ho + w, :]
        rec = jax.lax.dot_general(
            tok, d_r, (((1,), (1,)), ((), ())),
            preferred_element_type=jnp.float32)
        out_ref[:, a0:a0 + w] += rec


def kernel(spikes, neuron_regions, eids, enc_w, enc_b, dec_w, dec_b):
    B, T, N = spikes.shape
    M = B * T
    D = enc_w[0].shape[0]
    R = len(enc_w)
    sizes = [wt.shape[1] for wt in enc_w]
    offs = [0]
    for n in sizes:
        offs.append(offs[-1] + n)

    eh, dh, groups = [], [], []
    ho = 0
    for i in range(R):
        off, n = offs[i], sizes[i]
        a0 = (off // _LANE) * _LANE
        a1 = -(-(off + n) // _LANE) * _LANE
        w = a1 - a0
        lo, hi = off - a0, a1 - (off + n)
        eh.append(jnp.pad(enc_w[i], ((0, 0), (lo, hi))).astype(jnp.bfloat16))
        dh.append(jnp.pad(dec_w[i], ((lo, hi), (0, 0))).astype(jnp.bfloat16))
        groups.append((a0, w, ho))
        ho += w

    sp2 = spikes.reshape(M, N)
    E = jnp.concatenate(eh, axis=1)
    Dc = jnp.concatenate(dh, axis=0)
    EB = jnp.stack(enc_b, axis=0)
    DB = jnp.concatenate(dec_b)[None, :]
    W = E.shape[1]

    TM = 160
    grid = (M // TM,)
    out = pl.pallas_call(
        functools.partial(_codec_body, tuple(groups)),
        grid=grid,
        in_specs=[
            pl.BlockSpec((TM, N), lambda i: (i, 0)),
            pl.BlockSpec((D, W), lambda i: (0, 0)),
            pl.BlockSpec((W, D), lambda i: (0, 0)),
            pl.BlockSpec(EB.shape, lambda i: (0, 0)),
            pl.BlockSpec((1, N), lambda i: (0, 0)),
        ],
        out_specs=pl.BlockSpec((TM, N), lambda i: (i, 0)),
        out_shape=jax.ShapeDtypeStruct((M, N), spikes.dtype),
    )(sp2, E, Dc, EB, DB)
    return out.reshape(B, T, N)
